# Initial kernel scaffold; baseline (speedup 1.0000x reference)
import jax
import jax.numpy as jnp
from jax.experimental import pallas as pl


def kernel(feat, edge_index, edge_weight, beta, eps):
    raise NotImplementedError("write your pallas kernel here")


# speedup vs baseline: 8.6738x; 8.6738x over previous
import functools

import jax
import jax.numpy as jnp
from jax import lax
from jax.experimental import pallas as pl
from jax.experimental.pallas import tpu as pltpu
from jax.experimental.pallas import tpu_sc as plsc

_L = 16
_C = 128


def _copy_row(src1d, r, dst1d):
    for j in range(_C // _L):
        dst1d[pl.ds(j * _L, _L)] = src1d[pl.ds(r * _C + j * _L, _L)]


def _splat(vec, i):
    idx = jnp.full((_L, 1), i, jnp.int32)
    return lax.gather(
        vec, idx,
        lax.GatherDimensionNumbers(
            offset_dims=(), collapsed_slice_dims=(0,), start_index_map=(0,)),
        slice_sizes=(1,),
        mode=lax.GatherScatterMode.PROMISE_IN_BOUNDS)


def _edge_exp_body(ew_ref, beta_ref, ee_ref):
    e = beta_ref[...] * ew_ref[...]
    ee_ref[...] = jnp.exp(e - jnp.max(e))


def _gdiv_body(feat_ref, segp_ref, g_ref):
    f = feat_ref[...]
    nrm = jnp.sqrt(jnp.sum(f * f, axis=-1, keepdims=True))
    seg = segp_ref[0] + segp_ref[1]
    g_ref[...] = f / (jnp.maximum(nrm, 1e-12) * seg)


def _final_body(feat_ref, hp_ref, eps_ref, out_ref):
    out_ref[...] = (1.0 + eps_ref[...]) * feat_ref[...] + hp_ref[0] + hp_ref[1]


def _make_seg_sum(n_nodes, n_rows, nc, ns):
    n_tiles = nc * ns
    rpw = n_rows // n_tiles
    blk = 8
    n_blocks = rpw // blk
    rpt = (n_nodes // ns) & ~7
    tail = n_nodes - rpt * ns

    mesh = plsc.VectorSubcoreMesh(core_axis_name="c", subcore_axis_name="s")

    def body(src_hbm, ee_hbm, out_hbm,
             i0, i1, i2, i3, i4, i5, i6, i7,
             e0, e1, e2, e3, e4, e5, e6, e7, zb_v,
             semL, semS, acc_sh):
        c = lax.axis_index("c")
        s = lax.axis_index("s")
        wid = c * ns + s
        base = wid * rpw
        idx1 = (i0, i1, i2, i3, i4, i5, i6, i7)
        ee1 = (e0, e1, e2, e3, e4, e5, e6, e7)

        def zvec(i, carry):
            zb_v[pl.ds(i * _L, _L)] = jnp.zeros((_L,), jnp.float32)
            return carry

        lax.fori_loop(0, rpt // _L, zvec, 0)
        pltpu.sync_copy(zb_v, acc_sh.at[pl.ds(s * rpt, rpt)])

        @pl.when(s == ns - 1)
        def _():
            pltpu.sync_copy(zb_v.at[pl.ds(0, tail)],
                            acc_sh.at[pl.ds(ns * rpt, tail)])

        plsc.subcore_barrier()

        def scat_desc(r):
            return pltpu.make_async_copy(ee1[r], acc_sh.at[idx1[r]], semS)

        def i_desc(jb, r):
            off = (base + jb * blk) * _C
            return pltpu.make_async_copy(
                src_hbm.at[pl.ds(off + r * _C, _C)], idx1[r], semL)

        def e_desc(jb, r):
            off = (base + jb * blk) * _C
            return pltpu.make_async_copy(
                ee_hbm.at[pl.ds(off + r * _C, _C)], ee1[r], semL)

        def emit_block(jb, drain):
            if drain:
                for r in range(blk):
                    scat_desc(r).wait()
            for r in range(blk):
                i_desc(jb, r).start()
                e_desc(jb, r).start()
            for r in range(blk):
                i_desc(jb, r).wait()
                e_desc(jb, r).wait()
            for r in range(blk):
                scat_desc(r).start(add=True)

        emit_block(0, False)

        def outer(j, carry):
            emit_block(j, True)
            return carry

        lax.fori_loop(1, n_blocks, outer, 0)
        for r in range(blk):
            scat_desc(r).wait()

        plsc.subcore_barrier()
        pltpu.sync_copy(acc_sh.at[pl.ds(s * rpt, rpt)], zb_v)
        pltpu.sync_copy(zb_v, out_hbm.at[pl.ds(c * n_nodes + s * rpt, rpt)])

        @pl.when(s == ns - 1)
        def _():
            pltpu.sync_copy(acc_sh.at[pl.ds(ns * rpt, tail)],
                            zb_v.at[pl.ds(0, tail)])
            pltpu.sync_copy(
                zb_v.at[pl.ds(0, tail)],
                out_hbm.at[pl.ds(c * n_nodes + ns * rpt, tail)])

    return pl.kernel(
        body,
        out_type=jax.ShapeDtypeStruct((nc * n_nodes,), jnp.float32),
        mesh=mesh,
        scratch_types=(
            [pltpu.VMEM((_C,), jnp.int32)] * blk
            + [pltpu.VMEM((_C,), jnp.float32)] * blk
            + [pltpu.VMEM((rpt,), jnp.float32)]
            + [pltpu.SemaphoreType.DMA] * 2
            + [pltpu.VMEM_SHARED((n_nodes,), jnp.float32)]
        ),
    )


def _make_scatter(n_nodes, n_rows, d, nc, ns):
    n_tiles = nc * ns
    rpw = n_rows // n_tiles
    blk = 8
    n_blocks = rpw // blk
    rpt = (n_nodes // ns) & ~7
    tail = n_nodes - rpt * ns
    d_vecs = d // _L
    zrows = 104
    n_zchunks = rpt // zrows

    mesh = plsc.VectorSubcoreMesh(core_axis_name="c", subcore_axis_name="s")

    def body(src_hbm, dst_hbm, ee_hbm, g_hbm, out_hbm,
             eev_v, rows0, rows1, si0, si1, di0, di1, zbuf_v,
             semL, semI0, semI1, semD0, semD1, semG0, semG1,
             semS0, semS1, acc_sh):
        c = lax.axis_index("c")
        s = lax.axis_index("s")
        wid = c * ns + s
        base = wid * rpw
        rows = (rows0, rows1)
        sidx1 = (si0, si1)
        didx1 = (di0, di1)
        semI = (semI0, semI1)
        semD = (semD0, semD1)
        semG = (semG0, semG1)
        semS = (semS0, semS1)

        def zrow(i, carry):
            for j in range(d_vecs):
                zbuf_v[i, pl.ds(j * _L, _L)] = jnp.zeros((_L,), jnp.float32)
            return carry

        lax.fori_loop(0, zrows, zrow, 0)

        def zchunk(z, carry):
            pltpu.sync_copy(
                zbuf_v, acc_sh.at[pl.ds(s * rpt + z * zrows, zrows)])
            return carry

        lax.fori_loop(0, n_zchunks, zchunk, 0)

        @pl.when(s == ns - 1)
        def _():
            pltpu.sync_copy(zbuf_v.at[pl.ds(0, tail)],
                            acc_sh.at[pl.ds(ns * rpt, tail)])

        plsc.subcore_barrier()

        def g_desc(b):
            return pltpu.make_async_copy(
                g_hbm.at[sidx1[b]], rows[b], semG[b])

        def s_desc(b):
            return pltpu.make_async_copy(
                rows[b], acc_sh.at[didx1[b]], semS[b])

        def scale(r, b):
            rb = rows[b]

            def grp(gi, carry):
                vec = eev_v[pl.ds(r * _C + gi * _L, _L)]
                for i in range(_L):
                    sp = _splat(vec, i)

                    def mul16(row):
                        for j in range(d_vecs):
                            rb[row, pl.ds(j * _L, _L)] = (
                                rb[row, pl.ds(j * _L, _L)] * sp)
                    mul16(gi * _L + i)
                return carry

            lax.fori_loop(0, _C // _L, grp, 0)

        def i_desc(jb, r, b):
            off = (base + jb * blk) * _C
            return pltpu.make_async_copy(
                src_hbm.at[pl.ds(off + r * _C, _C)], sidx1[b], semI[b])

        def d_desc(jb, r, b):
            off = (base + jb * blk) * _C
            return pltpu.make_async_copy(
                dst_hbm.at[pl.ds(off + r * _C, _C)], didx1[b], semD[b])

        def block(jb, carry):
            off = (base + jb * blk) * _C

            @pl.when(jb >= 1)
            def _():
                s_desc(1).wait()

            pltpu.make_async_copy(
                ee_hbm.at[pl.ds(off, blk * _C)], eev_v, semL).start()
            i_desc(jb, 0, 0).start()
            d_desc(jb, 0, 0).start()
            pltpu.make_async_copy(
                ee_hbm.at[pl.ds(off, blk * _C)], eev_v, semL).wait()
            i_desc(jb, 0, 0).wait()
            g_desc(0).start()
            for r in range(blk):
                b = r % 2
                g_desc(b).wait()
                if r < blk - 1:
                    i_desc(jb, r + 1, 1 - b).start()
                scale(r, b)
                if r > 0:
                    s_desc(1 - b).wait()
                if r < blk - 1:
                    d_desc(jb, r + 1, 1 - b).start()
                    i_desc(jb, r + 1, 1 - b).wait()
                    g_desc(1 - b).start()
                d_desc(jb, r, b).wait()
                s_desc(b).start(add=True)
            return carry

        lax.fori_loop(0, n_blocks, block, 0)
        s_desc(1).wait()

        plsc.subcore_barrier()
        pltpu.sync_copy(acc_sh.at[pl.ds(s * rpt, rpt)],
                        out_hbm.at[c, pl.ds(s * rpt, rpt)])

        @pl.when(s == ns - 1)
        def _():
            pltpu.sync_copy(acc_sh.at[pl.ds(ns * rpt, tail)],
                            out_hbm.at[c, pl.ds(ns * rpt, tail)])

    return pl.kernel(
        body,
        out_type=jax.ShapeDtypeStruct((nc, n_nodes, d), jnp.float32),
        mesh=mesh,
        scratch_types=(
            [pltpu.VMEM((blk * _C,), jnp.float32)]
            + [pltpu.VMEM((_C, d), jnp.float32)] * 2
            + [pltpu.VMEM((_C,), jnp.int32)] * 4
            + [pltpu.VMEM((zrows, d), jnp.float32)]
            + [pltpu.SemaphoreType.DMA] * 9
            + [pltpu.VMEM_SHARED((n_nodes, d), jnp.float32)]
        ),
    )


def kernel(feat, edge_index, edge_weight, beta, eps):
    n, d = feat.shape
    n_edges = edge_weight.shape[0]
    src = edge_index[0].astype(jnp.int32)
    dst = edge_index[1].astype(jnp.int32)

    info = plsc.get_sparse_core_info()
    nc, ns = info.num_cores, info.num_subcores
    n_tiles = nc * ns

    unit = n_tiles * 8 * _C
    n_pad = ((n_edges + unit - 1) // unit) * unit
    n_rows = n_pad // _C

    ew2 = edge_weight.reshape(n_edges // _C, _C)
    ee2 = pl.pallas_call(
        _edge_exp_body,
        out_shape=jax.ShapeDtypeStruct(ew2.shape, jnp.float32),
    )(ew2, beta.reshape(1, 1))

    pad_rows = n_rows - n_edges // _C
    ee1 = jnp.pad(ee2, ((0, pad_rows), (0, 0))).reshape(-1)
    src1 = jnp.pad(src, (0, n_pad - n_edges))
    dst1 = jnp.pad(dst, (0, n_pad - n_edges))

    segp = _make_seg_sum(n, n_rows, nc, ns)(src1, ee1)

    g = pl.pallas_call(
        _gdiv_body,
        out_shape=jax.ShapeDtypeStruct((n, d), jnp.float32),
    )(feat, segp.reshape(nc, n, 1))

    hp = _make_scatter(n, n_rows, d, nc, ns)(src1, dst1, ee1, g)

    out = pl.pallas_call(
        _final_body,
        out_shape=jax.ShapeDtypeStruct((n, d), jnp.float32),
    )(feat, hp, eps.reshape(1, 1))
    return out
